# baseline (speedup 1.0000x reference)
import functools

import jax
import jax.numpy as jnp
from jax import lax
from jax.experimental import pallas as pl
from jax.experimental.pallas import tpu as pltpu
from jax.experimental.pallas import tpu_sc as plsc

N = 10000
E = 320000
F_IN = 128
DIM = 20
C = 2
G = 32
NUM_LAYERS = 3

DP = 32
NC = 2
NS = 16
NW = NC * NS
EPT = 10112
NP = 10112
STRIPE = NP // NS
NBUF = 2


def _make_sc_agg(width, chunk):
    nch = EPT // chunk

    def body(h_hbm, src_hbm, dst_hbm, zeros_hbm, out_hbm,
             src_v, dst_v, rows_v, acc):
        cid = lax.axis_index("c")
        sid = lax.axis_index("s")
        wid = cid * NS + sid
        pltpu.sync_copy(zeros_hbm, acc.at[pl.ds(sid * STRIPE, STRIPE)])
        pltpu.sync_copy(src_hbm.at[wid], src_v)
        pltpu.sync_copy(dst_hbm.at[wid], dst_v)
        plsc.subcore_barrier()

        def step(j, carry):
            pltpu.sync_copy(h_hbm.at[src_v.at[j]], rows_v)
            pltpu.sync_copy(rows_v, acc.at[dst_v.at[j]], add=True)
            return carry

        lax.fori_loop(0, nch, step, 0)
        plsc.subcore_barrier()
        pltpu.sync_copy(acc.at[pl.ds(sid * STRIPE, STRIPE)],
                        out_hbm.at[cid, pl.ds(sid * STRIPE, STRIPE)])

    mesh = plsc.VectorSubcoreMesh(core_axis_name="c", subcore_axis_name="s",
                                  num_cores=NC, num_subcores=NS)
    return pl.kernel(
        body,
        out_type=jax.ShapeDtypeStruct((NC, NP, width), jnp.float32),
        mesh=mesh,
        compiler_params=pltpu.CompilerParams(use_tc_tiling_on_sc=False),
        scratch_types=[
            pltpu.VMEM((nch, chunk), jnp.int32),
            pltpu.VMEM((nch, chunk), jnp.int32),
            pltpu.VMEM((chunk, width), jnp.float32),
            pltpu.VMEM_SHARED((NP, width), jnp.float32),
        ],
    )


def _bn(z, g_ref, b_ref):
    mean = jnp.mean(z, axis=0, keepdims=True)
    var = jnp.mean((z - mean) ** 2, axis=0, keepdims=True)
    return (z - mean) / jnp.sqrt(var + 1e-5) * g_ref[...] + b_ref[...]


def _mlp(z, w1_ref, b1_ref, g1_ref, bt1_ref, w2_ref, b2_ref, g2_ref, bt2_ref):
    z = jnp.dot(z, w1_ref[...], preferred_element_type=jnp.float32) + b1_ref[...]
    z = _bn(z, g1_ref, bt1_ref)
    z = jnp.dot(z, w2_ref[...], preferred_element_type=jnp.float32) + b2_ref[...]
    z = _bn(z, g2_ref, bt2_ref)
    return jnp.maximum(z, 0.0)


def _layer_body(p0_ref, p1_ref, h_ref, w1_ref, b1_ref, g1_ref, bt1_ref,
                w2_ref, b2_ref, g2_ref, bt2_ref, o_ref):
    z = p0_ref[...] + p1_ref[...] + h_ref[...]
    o_ref[...] = _mlp(z, w1_ref, b1_ref, g1_ref, bt1_ref,
                      w2_ref, b2_ref, g2_ref, bt2_ref)


def _final_body(p0_ref, p1_ref, h_ref, w1_ref, b1_ref, g1_ref, bt1_ref,
                w2_ref, b2_ref, g2_ref, bt2_ref,
                nw_ref, batch_ref, fcw_ref, fcb_ref,
                emb_ref, ge_ref, lg_ref):
    z = p0_ref[...] + p1_ref[...] + h_ref[...]
    h = _mlp(z, w1_ref, b1_ref, g1_ref, bt1_ref,
             w2_ref, b2_ref, g2_ref, bt2_ref)
    emb_ref[...] = h
    gids = lax.broadcasted_iota(jnp.int32, (N, G), 1)
    mask = (batch_ref[...] == gids).astype(jnp.float32)
    wg = mask * nw_ref[...]
    ge = lax.dot_general(wg, h, (((0,), (0,)), ((), ())),
                         preferred_element_type=jnp.float32,
                         precision=lax.Precision.HIGHEST)
    ge_ref[...] = ge
    lg_ref[...] = jnp.dot(ge, fcw_ref[...],
                          preferred_element_type=jnp.float32) + fcb_ref[...]


def _pad2(a, rows, cols):
    return jnp.zeros((rows, cols), jnp.float32).at[:a.shape[0], :a.shape[1]].set(a)


def _pad_row(v, cols):
    return jnp.zeros((1, cols), jnp.float32).at[0, :v.shape[0]].set(v)


def kernel(x, edge_index, batch, node_weight, params):
    src = jnp.zeros((NW * EPT,), jnp.int32).at[:E].set(edge_index[0])
    trash = N + (jnp.arange(NW * EPT, dtype=jnp.int32) % (NP - N))
    dst = trash.at[:E].set(edge_index[1])

    src_tw = src.reshape(NW, EPT // 128, 128)
    dst_tw = dst.reshape(NW, EPT // 128, 128)
    src_tn, dst_tn = src_tw, dst_tw
    zeros_wide = jnp.zeros((STRIPE, F_IN), jnp.float32)
    zeros_nar = jnp.zeros((STRIPE, DP), jnp.float32)

    lp = [params["layer%d" % i] for i in range(NUM_LAYERS)]
    w1 = [_pad2(lp[0]["W1"], F_IN, DP)] + \
         [_pad2(lp[i]["W1"], DP, DP) for i in range(1, NUM_LAYERS)]
    w2 = [_pad2(lp[i]["W2"], DP, DP) for i in range(NUM_LAYERS)]
    b1 = [_pad_row(lp[i]["b1"], DP) for i in range(NUM_LAYERS)]
    g1 = [_pad_row(lp[i]["g1"], DP) for i in range(NUM_LAYERS)]
    bt1 = [_pad_row(lp[i]["bt1"], DP) for i in range(NUM_LAYERS)]
    b2 = [_pad_row(lp[i]["b2"], DP) for i in range(NUM_LAYERS)]
    g2 = [_pad_row(lp[i]["g2"], DP) for i in range(NUM_LAYERS)]
    bt2 = [_pad_row(lp[i]["bt2"], DP) for i in range(NUM_LAYERS)]
    fcw = _pad2(params["fc_W"], DP, 128)
    fcb = _pad_row(params["fc_b"], 128)

    sc_agg_wide = _make_sc_agg(F_IN, 128)
    sc_agg_nar = _make_sc_agg(DP, 128)

    h = x
    for i in range(NUM_LAYERS):
        if i == 0:
            parts = sc_agg_wide(h, src_tw, dst_tw, zeros_wide)
        else:
            parts = sc_agg_nar(h, src_tn, dst_tn, zeros_nar)
        p0 = parts[0, :N]
        p1 = parts[1, :N]
        args = (p0, p1, h, w1[i], b1[i], g1[i], bt1[i],
                w2[i], b2[i], g2[i], bt2[i])
        if i < NUM_LAYERS - 1:
            h = pl.pallas_call(
                _layer_body,
                out_shape=jax.ShapeDtypeStruct((N, DP), jnp.float32),
            )(*args)
        else:
            emb, ge, lg = pl.pallas_call(
                _final_body,
                out_shape=[
                    jax.ShapeDtypeStruct((N, DP), jnp.float32),
                    jax.ShapeDtypeStruct((G, DP), jnp.float32),
                    jax.ShapeDtypeStruct((G, 128), jnp.float32),
                ],
            )(*args, node_weight.reshape(N, 1), batch.reshape(N, 1), fcw, fcb)

    node_emb = emb[:, :DIM]
    graph_emb = ge[:, :DIM]
    logits = lg[:, :C]
    return node_emb, graph_emb, logits


# speedup vs baseline: 1.2259x; 1.2259x over previous
import functools

import jax
import jax.numpy as jnp
from jax import lax
from jax.experimental import pallas as pl
from jax.experimental.pallas import tpu as pltpu
from jax.experimental.pallas import tpu_sc as plsc

N = 10000
E = 320000
F_IN = 128
DIM = 20
C = 2
G = 32
NUM_LAYERS = 3

DP = 32
NC = 2
NS = 16
NW = NC * NS
EPT = 10112
NP = 10112
STRIPE = NP // NS
NBUF = 2


def _make_sc_agg(width, chunk):
    nch = EPT // chunk

    def body(h_hbm, src_hbm, dst_hbm, zeros_hbm, out_hbm,
             src_v, dst_v, rows_v, acc):
        cid = lax.axis_index("c")
        sid = lax.axis_index("s")
        wid = cid * NS + sid
        pltpu.sync_copy(zeros_hbm, acc.at[pl.ds(sid * STRIPE, STRIPE)])
        pltpu.sync_copy(src_hbm.at[wid], src_v)
        pltpu.sync_copy(dst_hbm.at[wid], dst_v)
        plsc.subcore_barrier()

        def step(j, carry):
            pltpu.sync_copy(h_hbm.at[src_v.at[j]], rows_v)
            pltpu.sync_copy(rows_v, acc.at[dst_v.at[j]], add=True)
            return carry

        lax.fori_loop(0, nch, step, 0)
        plsc.subcore_barrier()
        pltpu.sync_copy(acc.at[pl.ds(sid * STRIPE, STRIPE)],
                        out_hbm.at[cid, pl.ds(sid * STRIPE, STRIPE)])

    mesh = plsc.VectorSubcoreMesh(core_axis_name="c", subcore_axis_name="s",
                                  num_cores=NC, num_subcores=NS)
    return pl.kernel(
        body,
        out_type=jax.ShapeDtypeStruct((NC, NP, width), jnp.float32),
        mesh=mesh,
        compiler_params=pltpu.CompilerParams(use_tc_tiling_on_sc=False),
        scratch_types=[
            pltpu.VMEM((nch, chunk), jnp.int32),
            pltpu.VMEM((nch, chunk), jnp.int32),
            pltpu.VMEM((chunk, width), jnp.float32),
            pltpu.VMEM_SHARED((NP, width), jnp.float32),
        ],
    )


def _make_sc_agg_staged(width, chunk):
    nch = EPT // chunk
    tstripe = N // NS

    def body(h_hbm, src_hbm, dst_hbm, zeros_hbm, out_hbm,
             src_v, dst_v, rows_v, table, acc):
        cid = lax.axis_index("c")
        sid = lax.axis_index("s")
        wid = cid * NS + sid
        pltpu.sync_copy(zeros_hbm, acc.at[pl.ds(sid * STRIPE, STRIPE)])
        pltpu.sync_copy(h_hbm.at[pl.ds(sid * tstripe, tstripe)],
                        table.at[pl.ds(sid * tstripe, tstripe)])
        pltpu.sync_copy(src_hbm.at[wid], src_v)
        pltpu.sync_copy(dst_hbm.at[wid], dst_v)
        plsc.subcore_barrier()

        def step(j, carry):
            pltpu.sync_copy(table.at[src_v.at[j]], rows_v)
            pltpu.sync_copy(rows_v, acc.at[dst_v.at[j]], add=True)
            return carry

        lax.fori_loop(0, nch, step, 0)
        plsc.subcore_barrier()
        pltpu.sync_copy(acc.at[pl.ds(sid * STRIPE, STRIPE)],
                        out_hbm.at[cid, pl.ds(sid * STRIPE, STRIPE)])

    mesh = plsc.VectorSubcoreMesh(core_axis_name="c", subcore_axis_name="s",
                                  num_cores=NC, num_subcores=NS)
    return pl.kernel(
        body,
        out_type=jax.ShapeDtypeStruct((NC, NP, width), jnp.float32),
        mesh=mesh,
        compiler_params=pltpu.CompilerParams(use_tc_tiling_on_sc=False),
        scratch_types=[
            pltpu.VMEM((nch, chunk), jnp.int32),
            pltpu.VMEM((nch, chunk), jnp.int32),
            pltpu.VMEM((chunk, width), jnp.float32),
            pltpu.VMEM_SHARED((N, width), jnp.float32),
            pltpu.VMEM_SHARED((NP, width), jnp.float32),
        ],
    )


def _bn(z, g_ref, b_ref):
    mean = jnp.mean(z, axis=0, keepdims=True)
    var = jnp.mean((z - mean) ** 2, axis=0, keepdims=True)
    return (z - mean) / jnp.sqrt(var + 1e-5) * g_ref[...] + b_ref[...]


def _mlp(z, w1_ref, b1_ref, g1_ref, bt1_ref, w2_ref, b2_ref, g2_ref, bt2_ref):
    z = jnp.dot(z, w1_ref[...], preferred_element_type=jnp.float32) + b1_ref[...]
    z = _bn(z, g1_ref, bt1_ref)
    z = jnp.dot(z, w2_ref[...], preferred_element_type=jnp.float32) + b2_ref[...]
    z = _bn(z, g2_ref, bt2_ref)
    return jnp.maximum(z, 0.0)


def _layer_body(p0_ref, p1_ref, h_ref, w1_ref, b1_ref, g1_ref, bt1_ref,
                w2_ref, b2_ref, g2_ref, bt2_ref, o_ref):
    z = p0_ref[...] + p1_ref[...] + h_ref[...]
    o_ref[...] = _mlp(z, w1_ref, b1_ref, g1_ref, bt1_ref,
                      w2_ref, b2_ref, g2_ref, bt2_ref)


def _final_body(p0_ref, p1_ref, h_ref, w1_ref, b1_ref, g1_ref, bt1_ref,
                w2_ref, b2_ref, g2_ref, bt2_ref,
                nw_ref, batch_ref, fcw_ref, fcb_ref,
                emb_ref, ge_ref, lg_ref):
    z = p0_ref[...] + p1_ref[...] + h_ref[...]
    h = _mlp(z, w1_ref, b1_ref, g1_ref, bt1_ref,
             w2_ref, b2_ref, g2_ref, bt2_ref)
    emb_ref[...] = h
    gids = lax.broadcasted_iota(jnp.int32, (N, G), 1)
    mask = (batch_ref[...] == gids).astype(jnp.float32)
    wg = mask * nw_ref[...]
    ge = lax.dot_general(wg, h, (((0,), (0,)), ((), ())),
                         preferred_element_type=jnp.float32,
                         precision=lax.Precision.HIGHEST)
    ge_ref[...] = ge
    lg_ref[...] = jnp.dot(ge, fcw_ref[...],
                          preferred_element_type=jnp.float32) + fcb_ref[...]


def _pad2(a, rows, cols):
    return jnp.zeros((rows, cols), jnp.float32).at[:a.shape[0], :a.shape[1]].set(a)


def _pad_row(v, cols):
    return jnp.zeros((1, cols), jnp.float32).at[0, :v.shape[0]].set(v)


def kernel(x, edge_index, batch, node_weight, params):
    src = jnp.zeros((NW * EPT,), jnp.int32).at[:E].set(edge_index[0])
    trash = N + (jnp.arange(NW * EPT, dtype=jnp.int32) % (NP - N))
    dst = trash.at[:E].set(edge_index[1])

    src_tw = src.reshape(NW, EPT // 128, 128)
    dst_tw = dst.reshape(NW, EPT // 128, 128)
    src_tn, dst_tn = src_tw, dst_tw
    zeros_wide = jnp.zeros((STRIPE, F_IN), jnp.float32)
    zeros_nar = jnp.zeros((STRIPE, DP), jnp.float32)

    lp = [params["layer%d" % i] for i in range(NUM_LAYERS)]
    w1 = [_pad2(lp[0]["W1"], F_IN, DP)] + \
         [_pad2(lp[i]["W1"], DP, DP) for i in range(1, NUM_LAYERS)]
    w2 = [_pad2(lp[i]["W2"], DP, DP) for i in range(NUM_LAYERS)]
    b1 = [_pad_row(lp[i]["b1"], DP) for i in range(NUM_LAYERS)]
    g1 = [_pad_row(lp[i]["g1"], DP) for i in range(NUM_LAYERS)]
    bt1 = [_pad_row(lp[i]["bt1"], DP) for i in range(NUM_LAYERS)]
    b2 = [_pad_row(lp[i]["b2"], DP) for i in range(NUM_LAYERS)]
    g2 = [_pad_row(lp[i]["g2"], DP) for i in range(NUM_LAYERS)]
    bt2 = [_pad_row(lp[i]["bt2"], DP) for i in range(NUM_LAYERS)]
    fcw = _pad2(params["fc_W"], DP, 128)
    fcb = _pad_row(params["fc_b"], 128)

    sc_agg_wide = _make_sc_agg(F_IN, 128)
    sc_agg_nar = _make_sc_agg_staged(DP, 128)

    h = x
    for i in range(NUM_LAYERS):
        if i == 0:
            parts = sc_agg_wide(h, src_tw, dst_tw, zeros_wide)
        else:
            parts = sc_agg_nar(h, src_tn, dst_tn, zeros_nar)
        p0 = parts[0, :N]
        p1 = parts[1, :N]
        args = (p0, p1, h, w1[i], b1[i], g1[i], bt1[i],
                w2[i], b2[i], g2[i], bt2[i])
        if i < NUM_LAYERS - 1:
            h = pl.pallas_call(
                _layer_body,
                out_shape=jax.ShapeDtypeStruct((N, DP), jnp.float32),
            )(*args)
        else:
            emb, ge, lg = pl.pallas_call(
                _final_body,
                out_shape=[
                    jax.ShapeDtypeStruct((N, DP), jnp.float32),
                    jax.ShapeDtypeStruct((G, DP), jnp.float32),
                    jax.ShapeDtypeStruct((G, 128), jnp.float32),
                ],
            )(*args, node_weight.reshape(N, 1), batch.reshape(N, 1), fcw, fcb)

    node_emb = emb[:, :DIM]
    graph_emb = ge[:, :DIM]
    logits = lg[:, :C]
    return node_emb, graph_emb, logits


# speedup vs baseline: 1.5373x; 1.2540x over previous
import functools

import jax
import jax.numpy as jnp
from jax import lax
from jax.experimental import pallas as pl
from jax.experimental.pallas import tpu as pltpu
from jax.experimental.pallas import tpu_sc as plsc

N = 10000
E = 320000
F_IN = 128
DIM = 20
C = 2
G = 32
NUM_LAYERS = 3

DP = 32
NC = 2
NS = 16
NW = NC * NS
EPT = 10112
NP = 10112
STRIPE = NP // NS
NBUF = 2


def _make_sc_agg(width, chunk):
    nch = EPT // chunk

    def body(h_hbm, src_hbm, dst_hbm, zeros_hbm, out_hbm,
             src_v, dst_v, rows_v, acc):
        cid = lax.axis_index("c")
        sid = lax.axis_index("s")
        wid = cid * NS + sid
        pltpu.sync_copy(zeros_hbm, acc.at[pl.ds(sid * STRIPE, STRIPE)])
        pltpu.sync_copy(src_hbm.at[wid], src_v)
        pltpu.sync_copy(dst_hbm.at[wid], dst_v)
        plsc.subcore_barrier()

        def step(j, carry):
            pltpu.sync_copy(h_hbm.at[src_v.at[j]], rows_v)
            pltpu.sync_copy(rows_v, acc.at[dst_v.at[j]], add=True)
            return carry

        lax.fori_loop(0, nch, step, 0)
        plsc.subcore_barrier()
        pltpu.sync_copy(acc.at[pl.ds(sid * STRIPE, STRIPE)],
                        out_hbm.at[cid, pl.ds(sid * STRIPE, STRIPE)])

    mesh = plsc.VectorSubcoreMesh(core_axis_name="c", subcore_axis_name="s",
                                  num_cores=NC, num_subcores=NS)
    return pl.kernel(
        body,
        out_type=jax.ShapeDtypeStruct((NC, NP, width), jnp.float32),
        mesh=mesh,
        compiler_params=pltpu.CompilerParams(use_tc_tiling_on_sc=False),
        scratch_types=[
            pltpu.VMEM((nch, chunk), jnp.int32),
            pltpu.VMEM((nch, chunk), jnp.int32),
            pltpu.VMEM((chunk, width), jnp.float32),
            pltpu.VMEM_SHARED((NP, width), jnp.float32),
        ],
    )


def _make_sc_agg_fsplit(chunk):
    half = F_IN // NC
    ept2 = NC * EPT
    nch = ept2 // chunk
    tstripe = N // NS

    def body(h_hbm, src_hbm, dst_hbm, zeros_hbm, out_hbm,
             src_v, dst_v, rows_v, table, acc):
        cid = lax.axis_index("c")
        sid = lax.axis_index("s")
        pltpu.sync_copy(zeros_hbm, acc.at[pl.ds(sid * STRIPE, STRIPE)])
        pltpu.sync_copy(
            h_hbm.at[pl.ds(sid * tstripe, tstripe), pl.ds(cid * half, half)],
            table.at[pl.ds(sid * tstripe, tstripe)])
        pltpu.sync_copy(src_hbm.at[sid], src_v)
        pltpu.sync_copy(dst_hbm.at[sid], dst_v)
        plsc.subcore_barrier()

        def step(j, carry):
            pltpu.sync_copy(table.at[src_v.at[j]], rows_v)
            pltpu.sync_copy(rows_v, acc.at[dst_v.at[j]], add=True)
            return carry

        lax.fori_loop(0, nch, step, 0)
        plsc.subcore_barrier()
        pltpu.sync_copy(acc.at[pl.ds(sid * STRIPE, STRIPE)],
                        out_hbm.at[cid, pl.ds(sid * STRIPE, STRIPE)])

    mesh = plsc.VectorSubcoreMesh(core_axis_name="c", subcore_axis_name="s",
                                  num_cores=NC, num_subcores=NS)
    return pl.kernel(
        body,
        out_type=jax.ShapeDtypeStruct((NC, NP, half), jnp.float32),
        mesh=mesh,
        compiler_params=pltpu.CompilerParams(use_tc_tiling_on_sc=False),
        scratch_types=[
            pltpu.VMEM((nch, chunk), jnp.int32),
            pltpu.VMEM((nch, chunk), jnp.int32),
            pltpu.VMEM((chunk, half), jnp.float32),
            pltpu.VMEM_SHARED((N, half), jnp.float32),
            pltpu.VMEM_SHARED((NP, half), jnp.float32),
        ],
    )


def _make_sc_agg_staged(width, chunk):
    nch = EPT // chunk
    tstripe = N // NS

    def body(h_hbm, src_hbm, dst_hbm, zeros_hbm, out_hbm,
             src_v, dst_v, rows_v, table, acc):
        cid = lax.axis_index("c")
        sid = lax.axis_index("s")
        wid = cid * NS + sid
        pltpu.sync_copy(zeros_hbm, acc.at[pl.ds(sid * STRIPE, STRIPE)])
        pltpu.sync_copy(h_hbm.at[pl.ds(sid * tstripe, tstripe)],
                        table.at[pl.ds(sid * tstripe, tstripe)])
        pltpu.sync_copy(src_hbm.at[wid], src_v)
        pltpu.sync_copy(dst_hbm.at[wid], dst_v)
        plsc.subcore_barrier()

        def step(j, carry):
            pltpu.sync_copy(table.at[src_v.at[j]], rows_v)
            pltpu.sync_copy(rows_v, acc.at[dst_v.at[j]], add=True)
            return carry

        lax.fori_loop(0, nch, step, 0)
        plsc.subcore_barrier()
        pltpu.sync_copy(acc.at[pl.ds(sid * STRIPE, STRIPE)],
                        out_hbm.at[cid, pl.ds(sid * STRIPE, STRIPE)])

    mesh = plsc.VectorSubcoreMesh(core_axis_name="c", subcore_axis_name="s",
                                  num_cores=NC, num_subcores=NS)
    return pl.kernel(
        body,
        out_type=jax.ShapeDtypeStruct((NC, NP, width), jnp.float32),
        mesh=mesh,
        compiler_params=pltpu.CompilerParams(use_tc_tiling_on_sc=False),
        scratch_types=[
            pltpu.VMEM((nch, chunk), jnp.int32),
            pltpu.VMEM((nch, chunk), jnp.int32),
            pltpu.VMEM((chunk, width), jnp.float32),
            pltpu.VMEM_SHARED((N, width), jnp.float32),
            pltpu.VMEM_SHARED((NP, width), jnp.float32),
        ],
    )


def _bn(z, g_ref, b_ref):
    mean = jnp.mean(z, axis=0, keepdims=True)
    var = jnp.mean((z - mean) ** 2, axis=0, keepdims=True)
    return (z - mean) / jnp.sqrt(var + 1e-5) * g_ref[...] + b_ref[...]


def _mlp(z, w1_ref, b1_ref, g1_ref, bt1_ref, w2_ref, b2_ref, g2_ref, bt2_ref):
    z = jnp.dot(z, w1_ref[...], preferred_element_type=jnp.float32) + b1_ref[...]
    z = _bn(z, g1_ref, bt1_ref)
    z = jnp.dot(z, w2_ref[...], preferred_element_type=jnp.float32) + b2_ref[...]
    z = _bn(z, g2_ref, bt2_ref)
    return jnp.maximum(z, 0.0)


def _layer_body(p0_ref, p1_ref, h_ref, w1_ref, b1_ref, g1_ref, bt1_ref,
                w2_ref, b2_ref, g2_ref, bt2_ref, o_ref):
    z = p0_ref[...] + p1_ref[...] + h_ref[...]
    o_ref[...] = _mlp(z, w1_ref, b1_ref, g1_ref, bt1_ref,
                      w2_ref, b2_ref, g2_ref, bt2_ref)


def _layer0_body(p0_ref, p1_ref, h_ref, w1_ref, b1_ref, g1_ref, bt1_ref,
                 w2_ref, b2_ref, g2_ref, bt2_ref, o_ref):
    agg = jnp.concatenate([p0_ref[...], p1_ref[...]], axis=1)
    z = agg + h_ref[...]
    o_ref[...] = _mlp(z, w1_ref, b1_ref, g1_ref, bt1_ref,
                      w2_ref, b2_ref, g2_ref, bt2_ref)


def _final_body(p0_ref, p1_ref, h_ref, w1_ref, b1_ref, g1_ref, bt1_ref,
                w2_ref, b2_ref, g2_ref, bt2_ref,
                nw_ref, batch_ref, fcw_ref, fcb_ref,
                emb_ref, ge_ref, lg_ref):
    z = p0_ref[...] + p1_ref[...] + h_ref[...]
    h = _mlp(z, w1_ref, b1_ref, g1_ref, bt1_ref,
             w2_ref, b2_ref, g2_ref, bt2_ref)
    emb_ref[...] = h
    gids = lax.broadcasted_iota(jnp.int32, (N, G), 1)
    mask = (batch_ref[...] == gids).astype(jnp.float32)
    wg = mask * nw_ref[...]
    ge = lax.dot_general(wg, h, (((0,), (0,)), ((), ())),
                         preferred_element_type=jnp.float32,
                         precision=lax.Precision.HIGHEST)
    ge_ref[...] = ge
    lg_ref[...] = jnp.dot(ge, fcw_ref[...],
                          preferred_element_type=jnp.float32) + fcb_ref[...]


def _pad2(a, rows, cols):
    return jnp.zeros((rows, cols), jnp.float32).at[:a.shape[0], :a.shape[1]].set(a)


def _pad_row(v, cols):
    return jnp.zeros((1, cols), jnp.float32).at[0, :v.shape[0]].set(v)


def kernel(x, edge_index, batch, node_weight, params):
    src = jnp.zeros((NW * EPT,), jnp.int32).at[:E].set(edge_index[0])
    trash = N + (jnp.arange(NW * EPT, dtype=jnp.int32) % (NP - N))
    dst = trash.at[:E].set(edge_index[1])

    src_tw = src.reshape(NS, NC * EPT // 128, 128)
    dst_tw = dst.reshape(NS, NC * EPT // 128, 128)
    src_tn = src.reshape(NW, EPT // 128, 128)
    dst_tn = dst.reshape(NW, EPT // 128, 128)
    zeros_half = jnp.zeros((STRIPE, F_IN // NC), jnp.float32)
    zeros_nar = jnp.zeros((STRIPE, DP), jnp.float32)

    lp = [params["layer%d" % i] for i in range(NUM_LAYERS)]
    w1 = [_pad2(lp[0]["W1"], F_IN, DP)] + \
         [_pad2(lp[i]["W1"], DP, DP) for i in range(1, NUM_LAYERS)]
    w2 = [_pad2(lp[i]["W2"], DP, DP) for i in range(NUM_LAYERS)]
    b1 = [_pad_row(lp[i]["b1"], DP) for i in range(NUM_LAYERS)]
    g1 = [_pad_row(lp[i]["g1"], DP) for i in range(NUM_LAYERS)]
    bt1 = [_pad_row(lp[i]["bt1"], DP) for i in range(NUM_LAYERS)]
    b2 = [_pad_row(lp[i]["b2"], DP) for i in range(NUM_LAYERS)]
    g2 = [_pad_row(lp[i]["g2"], DP) for i in range(NUM_LAYERS)]
    bt2 = [_pad_row(lp[i]["bt2"], DP) for i in range(NUM_LAYERS)]
    fcw = _pad2(params["fc_W"], DP, 128)
    fcb = _pad_row(params["fc_b"], 128)

    sc_agg_wide = _make_sc_agg_fsplit(128)
    sc_agg_nar = _make_sc_agg_staged(DP, 128)

    h = x
    for i in range(NUM_LAYERS):
        if i == 0:
            parts = sc_agg_wide(h, src_tw, dst_tw, zeros_half)
        else:
            parts = sc_agg_nar(h, src_tn, dst_tn, zeros_nar)
        p0 = parts[0, :N]
        p1 = parts[1, :N]
        args = (p0, p1, h, w1[i], b1[i], g1[i], bt1[i],
                w2[i], b2[i], g2[i], bt2[i])
        if i < NUM_LAYERS - 1:
            h = pl.pallas_call(
                _layer0_body if i == 0 else _layer_body,
                out_shape=jax.ShapeDtypeStruct((N, DP), jnp.float32),
            )(*args)
        else:
            emb, ge, lg = pl.pallas_call(
                _final_body,
                out_shape=[
                    jax.ShapeDtypeStruct((N, DP), jnp.float32),
                    jax.ShapeDtypeStruct((G, DP), jnp.float32),
                    jax.ShapeDtypeStruct((G, 128), jnp.float32),
                ],
            )(*args, node_weight.reshape(N, 1), batch.reshape(N, 1), fcw, fcb)

    node_emb = emb[:, :DIM]
    graph_emb = ge[:, :DIM]
    logits = lg[:, :C]
    return node_emb, graph_emb, logits


# speedup vs baseline: 1.6357x; 1.0641x over previous
import functools

import jax
import jax.numpy as jnp
from jax import lax
from jax.experimental import pallas as pl
from jax.experimental.pallas import tpu as pltpu
from jax.experimental.pallas import tpu_sc as plsc

N = 10000
E = 320000
F_IN = 128
DIM = 20
C = 2
G = 32
NUM_LAYERS = 3

DP = 32
NC = 2
NS = 16
NW = NC * NS
EPT = 10112
NP = 10112
STRIPE = NP // NS
NBUF = 2


def _make_sc_agg(width, chunk):
    nch = EPT // chunk

    def body(h_hbm, src_hbm, dst_hbm, zeros_hbm, out_hbm,
             src_v, dst_v, rows_v, acc):
        cid = lax.axis_index("c")
        sid = lax.axis_index("s")
        wid = cid * NS + sid
        pltpu.sync_copy(zeros_hbm, acc.at[pl.ds(sid * STRIPE, STRIPE)])
        pltpu.sync_copy(src_hbm.at[wid], src_v)
        pltpu.sync_copy(dst_hbm.at[wid], dst_v)
        plsc.subcore_barrier()

        def step(j, carry):
            pltpu.sync_copy(h_hbm.at[src_v.at[j]], rows_v)
            pltpu.sync_copy(rows_v, acc.at[dst_v.at[j]], add=True)
            return carry

        lax.fori_loop(0, nch, step, 0)
        plsc.subcore_barrier()
        pltpu.sync_copy(acc.at[pl.ds(sid * STRIPE, STRIPE)],
                        out_hbm.at[cid, pl.ds(sid * STRIPE, STRIPE)])

    mesh = plsc.VectorSubcoreMesh(core_axis_name="c", subcore_axis_name="s",
                                  num_cores=NC, num_subcores=NS)
    return pl.kernel(
        body,
        out_type=jax.ShapeDtypeStruct((NC, NP, width), jnp.float32),
        mesh=mesh,
        compiler_params=pltpu.CompilerParams(use_tc_tiling_on_sc=False),
        scratch_types=[
            pltpu.VMEM((nch, chunk), jnp.int32),
            pltpu.VMEM((nch, chunk), jnp.int32),
            pltpu.VMEM((chunk, width), jnp.float32),
            pltpu.VMEM_SHARED((NP, width), jnp.float32),
        ],
    )


def _make_sc_agg_fsplit(chunk):
    half = F_IN // NC
    ept2 = NC * EPT
    nch = ept2 // chunk
    tstripe = N // NS

    def body(h_hbm, src_hbm, dst_hbm, zeros_hbm, out_hbm,
             src_v, dst_v, rows_v, table, acc):
        cid = lax.axis_index("c")
        sid = lax.axis_index("s")
        pltpu.sync_copy(zeros_hbm, acc.at[pl.ds(sid * STRIPE, STRIPE)])
        pltpu.sync_copy(
            h_hbm.at[pl.ds(sid * tstripe, tstripe), pl.ds(cid * half, half)],
            table.at[pl.ds(sid * tstripe, tstripe)])
        pltpu.sync_copy(src_hbm.at[sid], src_v)
        pltpu.sync_copy(dst_hbm.at[sid], dst_v)
        plsc.subcore_barrier()

        def step(j, carry):
            pltpu.sync_copy(table.at[src_v.at[j]], rows_v)
            pltpu.sync_copy(rows_v, acc.at[dst_v.at[j]], add=True)
            return carry

        lax.fori_loop(0, nch, step, 0)
        plsc.subcore_barrier()
        pltpu.sync_copy(acc.at[pl.ds(sid * STRIPE, STRIPE)],
                        out_hbm.at[cid, pl.ds(sid * STRIPE, STRIPE)])

    mesh = plsc.VectorSubcoreMesh(core_axis_name="c", subcore_axis_name="s",
                                  num_cores=NC, num_subcores=NS)
    return pl.kernel(
        body,
        out_type=jax.ShapeDtypeStruct((NC, NP, half), jnp.float32),
        mesh=mesh,
        compiler_params=pltpu.CompilerParams(use_tc_tiling_on_sc=False),
        scratch_types=[
            pltpu.VMEM((nch, chunk), jnp.int32),
            pltpu.VMEM((nch, chunk), jnp.int32),
            pltpu.VMEM((chunk, half), jnp.float32),
            pltpu.VMEM_SHARED((N, half), jnp.float32),
            pltpu.VMEM_SHARED((NP, half), jnp.float32),
        ],
    )


def _make_sc_agg_staged(width, chunk):
    nch = EPT // chunk
    tstripe = N // NS

    def body(h_hbm, src_hbm, dst_hbm, zeros_hbm, out_hbm,
             src_v, dst_v, rows_v, table, acc):
        cid = lax.axis_index("c")
        sid = lax.axis_index("s")
        wid = cid * NS + sid
        pltpu.sync_copy(zeros_hbm, acc.at[pl.ds(sid * STRIPE, STRIPE)])
        pltpu.sync_copy(h_hbm.at[pl.ds(sid * tstripe, tstripe)],
                        table.at[pl.ds(sid * tstripe, tstripe)])
        pltpu.sync_copy(src_hbm.at[wid], src_v)
        pltpu.sync_copy(dst_hbm.at[wid], dst_v)
        plsc.subcore_barrier()

        def step(j, carry):
            pltpu.sync_copy(table.at[src_v.at[j]], rows_v)
            pltpu.sync_copy(rows_v, acc.at[dst_v.at[j]], add=True)
            return carry

        lax.fori_loop(0, nch, step, 0)
        plsc.subcore_barrier()
        pltpu.sync_copy(acc.at[pl.ds(sid * STRIPE, STRIPE)],
                        out_hbm.at[cid, pl.ds(sid * STRIPE, STRIPE)])

    mesh = plsc.VectorSubcoreMesh(core_axis_name="c", subcore_axis_name="s",
                                  num_cores=NC, num_subcores=NS)
    return pl.kernel(
        body,
        out_type=jax.ShapeDtypeStruct((NC, NP, width), jnp.float32),
        mesh=mesh,
        compiler_params=pltpu.CompilerParams(use_tc_tiling_on_sc=False),
        scratch_types=[
            pltpu.VMEM((nch, chunk), jnp.int32),
            pltpu.VMEM((nch, chunk), jnp.int32),
            pltpu.VMEM((chunk, width), jnp.float32),
            pltpu.VMEM_SHARED((N, width), jnp.float32),
            pltpu.VMEM_SHARED((NP, width), jnp.float32),
        ],
    )


def _bn(z, g_ref, b_ref):
    mean = jnp.mean(z, axis=0, keepdims=True)
    var = jnp.mean((z - mean) ** 2, axis=0, keepdims=True)
    return (z - mean) / jnp.sqrt(var + 1e-5) * g_ref[...] + b_ref[...]


def _mlp(z, w1_ref, b1_ref, g1_ref, bt1_ref, w2_ref, b2_ref, g2_ref, bt2_ref):
    z = jnp.dot(z, w1_ref[...], preferred_element_type=jnp.float32) + b1_ref[...]
    z = _bn(z, g1_ref, bt1_ref)
    z = jnp.dot(z, w2_ref[...], preferred_element_type=jnp.float32) + b2_ref[...]
    z = _bn(z, g2_ref, bt2_ref)
    return jnp.maximum(z, 0.0)


def _pad_w(w_ref):
    w = w_ref[...]
    r, c = w.shape
    w = jnp.concatenate([w, jnp.zeros((r, DP - c), jnp.float32)], axis=1)
    if r < DP:
        w = jnp.concatenate([w, jnp.zeros((DP - r, DP), jnp.float32)], axis=0)
    return w


def _pad_v(v_ref):
    return jnp.concatenate(
        [v_ref[...], jnp.zeros((1, DP - DIM), jnp.float32)], axis=1)


def _mlp_raw(z, w1_ref, b1_ref, g1_ref, bt1_ref, w2_ref, b2_ref, g2_ref,
             bt2_ref):
    z = jnp.dot(z, _pad_w(w1_ref),
                preferred_element_type=jnp.float32) + _pad_v(b1_ref)
    z = _bn(z, _pad_v(g1_ref), _pad_v(bt1_ref))
    z = jnp.dot(z, _pad_w(w2_ref),
                preferred_element_type=jnp.float32) + _pad_v(b2_ref)
    z = _bn(z, _pad_v(g2_ref), _pad_v(bt2_ref))
    return jnp.maximum(z, 0.0)


def _layer_body(pp_ref, h_ref, w1_ref, b1_ref, g1_ref, bt1_ref,
                w2_ref, b2_ref, g2_ref, bt2_ref, o_ref):
    z = pp_ref[0] + pp_ref[1] + h_ref[...]
    o_ref[...] = _mlp_raw(z, w1_ref, b1_ref, g1_ref, bt1_ref,
                          w2_ref, b2_ref, g2_ref, bt2_ref)


def _layer0_body(pp_ref, h_ref, w1_ref, b1_ref, g1_ref, bt1_ref,
                 w2_ref, b2_ref, g2_ref, bt2_ref, o_ref):
    agg = jnp.concatenate([pp_ref[0], pp_ref[1]], axis=1)
    z = agg + h_ref[...]
    o_ref[...] = _mlp_raw(z, w1_ref, b1_ref, g1_ref, bt1_ref,
                          w2_ref, b2_ref, g2_ref, bt2_ref)


def _final_body(pp_ref, h_ref, w1_ref, b1_ref, g1_ref, bt1_ref,
                w2_ref, b2_ref, g2_ref, bt2_ref,
                nw_ref, batch_ref, fcw_ref, fcb_ref,
                emb_ref, ge_ref, lg_ref):
    z = pp_ref[0] + pp_ref[1] + h_ref[...]
    h = _mlp_raw(z, w1_ref, b1_ref, g1_ref, bt1_ref,
                 w2_ref, b2_ref, g2_ref, bt2_ref)
    emb_ref[...] = h
    gids = lax.broadcasted_iota(jnp.int32, (N, G), 1)
    mask = (batch_ref[...] == gids).astype(jnp.float32)
    wg = mask * nw_ref[...]
    ge = lax.dot_general(wg, h, (((0,), (0,)), ((), ())),
                         preferred_element_type=jnp.float32,
                         precision=lax.Precision.HIGHEST)
    ge_ref[...] = ge
    fcw = jnp.concatenate(
        [fcw_ref[...], jnp.zeros((DIM, DP - C), jnp.float32)], axis=1)
    fcw = jnp.concatenate([fcw, jnp.zeros((DP - DIM, DP), jnp.float32)],
                          axis=0)
    fcb = jnp.concatenate(
        [fcb_ref[...], jnp.zeros((1, DP - C), jnp.float32)], axis=1)
    lg_ref[...] = jnp.dot(ge, fcw,
                          preferred_element_type=jnp.float32) + fcb


def _pad2(a, rows, cols):
    return jnp.zeros((rows, cols), jnp.float32).at[:a.shape[0], :a.shape[1]].set(a)


def _pad_row(v, cols):
    return jnp.zeros((1, cols), jnp.float32).at[0, :v.shape[0]].set(v)


def kernel(x, edge_index, batch, node_weight, params):
    npad = NW * EPT - E
    src = jnp.concatenate([edge_index[0], jnp.zeros((npad,), jnp.int32)])
    trash = N + (jnp.arange(npad, dtype=jnp.int32) % (NP - N))
    dst = jnp.concatenate([edge_index[1], trash])
    src_tw = src.reshape(NS, NC * EPT // 128, 128)
    dst_tw = dst.reshape(NS, NC * EPT // 128, 128)
    src_tn = src.reshape(NW, EPT // 128, 128)
    dst_tn = dst.reshape(NW, EPT // 128, 128)
    zeros_half = jnp.zeros((STRIPE, F_IN // NC), jnp.float32)
    zeros_nar = jnp.zeros((STRIPE, DP), jnp.float32)

    lp = [params["layer%d" % i] for i in range(NUM_LAYERS)]

    sc_agg_wide = _make_sc_agg_fsplit(128)
    sc_agg_nar = _make_sc_agg_staged(DP, 128)

    def pspec(w):
        return pl.BlockSpec((NC, N, w), lambda i: (0, 0, 0))

    def row(v):
        return v.reshape(1, -1)

    h = x
    for i in range(NUM_LAYERS):
        p = lp[i]
        if i == 0:
            parts = sc_agg_wide(h, src_tw, dst_tw, zeros_half)
            pw = F_IN // NC
        else:
            parts = sc_agg_nar(h, src_tn, dst_tn, zeros_nar)
            pw = DP
        args = (parts, h, p["W1"], row(p["b1"]), row(p["g1"]), row(p["bt1"]),
                p["W2"], row(p["b2"]), row(p["g2"]), row(p["bt2"]))
        nargs = len(args)
        in_specs = [pspec(pw)] + [
            pl.BlockSpec(a.shape, lambda i, nd=a.ndim: (0,) * nd)
            for a in args[1:]]
        if i < NUM_LAYERS - 1:
            h = pl.pallas_call(
                _layer0_body if i == 0 else _layer_body,
                grid=(1,),
                in_specs=in_specs,
                out_specs=pl.BlockSpec((N, DP), lambda i: (0, 0)),
                out_shape=jax.ShapeDtypeStruct((N, DP), jnp.float32),
            )(*args)
        else:
            extra = (node_weight.reshape(N, 1), batch.reshape(N, 1),
                     params["fc_W"], row(params["fc_b"]))
            in_specs += [pl.BlockSpec(a.shape, lambda i, nd=a.ndim: (0,) * nd)
                         for a in extra]
            emb, ge, lg = pl.pallas_call(
                _final_body,
                grid=(1,),
                in_specs=in_specs,
                out_specs=[pl.BlockSpec((N, DP), lambda i: (0, 0)),
                           pl.BlockSpec((G, DP), lambda i: (0, 0)),
                           pl.BlockSpec((G, DP), lambda i: (0, 0))],
                out_shape=[
                    jax.ShapeDtypeStruct((N, DP), jnp.float32),
                    jax.ShapeDtypeStruct((G, DP), jnp.float32),
                    jax.ShapeDtypeStruct((G, DP), jnp.float32),
                ],
            )(*args, *extra)

    node_emb = emb[:, :DIM]
    graph_emb = ge[:, :DIM]
    logits = lg[:, :C]
    return node_emb, graph_emb, logits


# speedup vs baseline: 1.7024x; 1.0408x over previous
import functools

import jax
import jax.numpy as jnp
from jax import lax
from jax.experimental import pallas as pl
from jax.experimental.pallas import tpu as pltpu
from jax.experimental.pallas import tpu_sc as plsc

N = 10000
E = 320000
F_IN = 128
DIM = 20
C = 2
G = 32
NUM_LAYERS = 3

DP = 24
NC = 2
NS = 16
NW = NC * NS
EPT = 10112
NP = 10112
STRIPE = NP // NS
NBUF = 2


def _make_sc_agg(width, chunk):
    nch = EPT // chunk

    def body(h_hbm, src_hbm, dst_hbm, zeros_hbm, out_hbm,
             src_v, dst_v, rows_v, acc):
        cid = lax.axis_index("c")
        sid = lax.axis_index("s")
        wid = cid * NS + sid
        pltpu.sync_copy(zeros_hbm, acc.at[pl.ds(sid * STRIPE, STRIPE)])
        pltpu.sync_copy(src_hbm.at[wid], src_v)
        pltpu.sync_copy(dst_hbm.at[wid], dst_v)
        plsc.subcore_barrier()

        def step(j, carry):
            pltpu.sync_copy(h_hbm.at[src_v.at[j]], rows_v)
            pltpu.sync_copy(rows_v, acc.at[dst_v.at[j]], add=True)
            return carry

        lax.fori_loop(0, nch, step, 0)
        plsc.subcore_barrier()
        pltpu.sync_copy(acc.at[pl.ds(sid * STRIPE, STRIPE)],
                        out_hbm.at[cid, pl.ds(sid * STRIPE, STRIPE)])

    mesh = plsc.VectorSubcoreMesh(core_axis_name="c", subcore_axis_name="s",
                                  num_cores=NC, num_subcores=NS)
    return pl.kernel(
        body,
        out_type=jax.ShapeDtypeStruct((NC, NP, width), jnp.float32),
        mesh=mesh,
        compiler_params=pltpu.CompilerParams(use_tc_tiling_on_sc=False),
        scratch_types=[
            pltpu.VMEM((nch, chunk), jnp.int32),
            pltpu.VMEM((nch, chunk), jnp.int32),
            pltpu.VMEM((chunk, width), jnp.float32),
            pltpu.VMEM_SHARED((NP, width), jnp.float32),
        ],
    )


def _make_sc_agg_fsplit(chunk):
    half = F_IN // NC
    ept2 = NC * EPT
    nch = ept2 // chunk
    tstripe = N // NS

    def body(h_hbm, src_hbm, dst_hbm, zeros_hbm, out_hbm,
             src_v, dst_v, rows_v, table, acc):
        cid = lax.axis_index("c")
        sid = lax.axis_index("s")
        pltpu.sync_copy(zeros_hbm, acc.at[pl.ds(sid * STRIPE, STRIPE)])
        pltpu.sync_copy(
            h_hbm.at[pl.ds(sid * tstripe, tstripe), pl.ds(cid * half, half)],
            table.at[pl.ds(sid * tstripe, tstripe)])
        pltpu.sync_copy(src_hbm.at[sid], src_v)
        pltpu.sync_copy(dst_hbm.at[sid], dst_v)
        plsc.subcore_barrier()

        def step(j, carry):
            pltpu.sync_copy(table.at[src_v.at[j]], rows_v)
            pltpu.sync_copy(rows_v, acc.at[dst_v.at[j]], add=True)
            return carry

        lax.fori_loop(0, nch, step, 0)
        plsc.subcore_barrier()
        pltpu.sync_copy(acc.at[pl.ds(sid * STRIPE, STRIPE)],
                        out_hbm.at[cid, pl.ds(sid * STRIPE, STRIPE)])

    mesh = plsc.VectorSubcoreMesh(core_axis_name="c", subcore_axis_name="s",
                                  num_cores=NC, num_subcores=NS)
    return pl.kernel(
        body,
        out_type=jax.ShapeDtypeStruct((NC, NP, half), jnp.float32),
        mesh=mesh,
        compiler_params=pltpu.CompilerParams(use_tc_tiling_on_sc=False),
        scratch_types=[
            pltpu.VMEM((nch, chunk), jnp.int32),
            pltpu.VMEM((nch, chunk), jnp.int32),
            pltpu.VMEM((chunk, half), jnp.float32),
            pltpu.VMEM_SHARED((N, half), jnp.float32),
            pltpu.VMEM_SHARED((NP, half), jnp.float32),
        ],
    )


def _make_sc_agg_staged(width, chunk):
    nch = EPT // chunk
    tstripe = N // NS

    def body(h_hbm, src_hbm, dst_hbm, zeros_hbm, out_hbm,
             src_v, dst_v, rows_v, table, acc):
        cid = lax.axis_index("c")
        sid = lax.axis_index("s")
        wid = cid * NS + sid
        pltpu.sync_copy(zeros_hbm, acc.at[pl.ds(sid * STRIPE, STRIPE)])
        pltpu.sync_copy(h_hbm.at[pl.ds(sid * tstripe, tstripe)],
                        table.at[pl.ds(sid * tstripe, tstripe)])
        pltpu.sync_copy(src_hbm.at[wid], src_v)
        pltpu.sync_copy(dst_hbm.at[wid], dst_v)
        plsc.subcore_barrier()

        def step(j, carry):
            pltpu.sync_copy(table.at[src_v.at[j]], rows_v)
            pltpu.sync_copy(rows_v, acc.at[dst_v.at[j]], add=True)
            return carry

        lax.fori_loop(0, nch, step, 0)
        plsc.subcore_barrier()
        pltpu.sync_copy(acc.at[pl.ds(sid * STRIPE, STRIPE)],
                        out_hbm.at[cid, pl.ds(sid * STRIPE, STRIPE)])

    mesh = plsc.VectorSubcoreMesh(core_axis_name="c", subcore_axis_name="s",
                                  num_cores=NC, num_subcores=NS)
    return pl.kernel(
        body,
        out_type=jax.ShapeDtypeStruct((NC, NP, width), jnp.float32),
        mesh=mesh,
        compiler_params=pltpu.CompilerParams(use_tc_tiling_on_sc=False),
        scratch_types=[
            pltpu.VMEM((nch, chunk), jnp.int32),
            pltpu.VMEM((nch, chunk), jnp.int32),
            pltpu.VMEM((chunk, width), jnp.float32),
            pltpu.VMEM_SHARED((N, width), jnp.float32),
            pltpu.VMEM_SHARED((NP, width), jnp.float32),
        ],
    )


def _bn(z, g_ref, b_ref):
    mean = jnp.mean(z, axis=0, keepdims=True)
    var = jnp.mean((z - mean) ** 2, axis=0, keepdims=True)
    return (z - mean) / jnp.sqrt(var + 1e-5) * g_ref[...] + b_ref[...]


def _mlp(z, w1_ref, b1_ref, g1_ref, bt1_ref, w2_ref, b2_ref, g2_ref, bt2_ref):
    z = jnp.dot(z, w1_ref[...], preferred_element_type=jnp.float32) + b1_ref[...]
    z = _bn(z, g1_ref, bt1_ref)
    z = jnp.dot(z, w2_ref[...], preferred_element_type=jnp.float32) + b2_ref[...]
    z = _bn(z, g2_ref, bt2_ref)
    return jnp.maximum(z, 0.0)


def _pad_w(w_ref):
    w = w_ref[...]
    r, c = w.shape
    w = jnp.concatenate([w, jnp.zeros((r, DP - c), jnp.float32)], axis=1)
    if r < DP:
        w = jnp.concatenate([w, jnp.zeros((DP - r, DP), jnp.float32)], axis=0)
    return w


def _pad_v(v_ref):
    return jnp.concatenate(
        [v_ref[...], jnp.zeros((1, DP - DIM), jnp.float32)], axis=1)


def _mlp_raw(z, w1_ref, b1_ref, g1_ref, bt1_ref, w2_ref, b2_ref, g2_ref,
             bt2_ref):
    z = jnp.dot(z, _pad_w(w1_ref),
                preferred_element_type=jnp.float32) + _pad_v(b1_ref)
    z = _bn(z, _pad_v(g1_ref), _pad_v(bt1_ref))
    z = jnp.dot(z, _pad_w(w2_ref),
                preferred_element_type=jnp.float32) + _pad_v(b2_ref)
    z = _bn(z, _pad_v(g2_ref), _pad_v(bt2_ref))
    return jnp.maximum(z, 0.0)


def _layer_body(pp_ref, h_ref, w1_ref, b1_ref, g1_ref, bt1_ref,
                w2_ref, b2_ref, g2_ref, bt2_ref, o_ref):
    z = pp_ref[0] + pp_ref[1] + h_ref[...]
    o_ref[...] = _mlp_raw(z, w1_ref, b1_ref, g1_ref, bt1_ref,
                          w2_ref, b2_ref, g2_ref, bt2_ref)


def _layer0_body(pp_ref, h_ref, w1_ref, b1_ref, g1_ref, bt1_ref,
                 w2_ref, b2_ref, g2_ref, bt2_ref, o_ref):
    agg = jnp.concatenate([pp_ref[0], pp_ref[1]], axis=1)
    z = agg + h_ref[...]
    o_ref[...] = _mlp_raw(z, w1_ref, b1_ref, g1_ref, bt1_ref,
                          w2_ref, b2_ref, g2_ref, bt2_ref)


def _final_body(pp_ref, h_ref, w1_ref, b1_ref, g1_ref, bt1_ref,
                w2_ref, b2_ref, g2_ref, bt2_ref,
                nw_ref, batch_ref, fcw_ref, fcb_ref,
                emb_ref, ge_ref, lg_ref):
    z = pp_ref[0] + pp_ref[1] + h_ref[...]
    h = _mlp_raw(z, w1_ref, b1_ref, g1_ref, bt1_ref,
                 w2_ref, b2_ref, g2_ref, bt2_ref)
    emb_ref[...] = h
    gids = lax.broadcasted_iota(jnp.int32, (N, G), 1)
    mask = (batch_ref[...] == gids).astype(jnp.float32)
    wg = mask * nw_ref[...]
    ge = lax.dot_general(wg, h, (((0,), (0,)), ((), ())),
                         preferred_element_type=jnp.float32,
                         precision=lax.Precision.HIGHEST)
    ge_ref[...] = ge
    fcw = jnp.concatenate(
        [fcw_ref[...], jnp.zeros((DIM, DP - C), jnp.float32)], axis=1)
    fcw = jnp.concatenate([fcw, jnp.zeros((DP - DIM, DP), jnp.float32)],
                          axis=0)
    fcb = jnp.concatenate(
        [fcb_ref[...], jnp.zeros((1, DP - C), jnp.float32)], axis=1)
    lg_ref[...] = jnp.dot(ge, fcw,
                          preferred_element_type=jnp.float32) + fcb


def _pad2(a, rows, cols):
    return jnp.zeros((rows, cols), jnp.float32).at[:a.shape[0], :a.shape[1]].set(a)


def _pad_row(v, cols):
    return jnp.zeros((1, cols), jnp.float32).at[0, :v.shape[0]].set(v)


def kernel(x, edge_index, batch, node_weight, params):
    npad = NW * EPT - E
    src = jnp.concatenate([edge_index[0], jnp.zeros((npad,), jnp.int32)])
    trash = N + (jnp.arange(npad, dtype=jnp.int32) % (NP - N))
    dst = jnp.concatenate([edge_index[1], trash])
    src_tw = src.reshape(NS, NC * EPT // 128, 128)
    dst_tw = dst.reshape(NS, NC * EPT // 128, 128)
    src_tn = src.reshape(NW, EPT // 128, 128)
    dst_tn = dst.reshape(NW, EPT // 128, 128)
    zeros_half = jnp.zeros((STRIPE, F_IN // NC), jnp.float32)
    zeros_nar = jnp.zeros((STRIPE, DP), jnp.float32)

    lp = [params["layer%d" % i] for i in range(NUM_LAYERS)]

    sc_agg_wide = _make_sc_agg_fsplit(128)
    sc_agg_nar = _make_sc_agg_staged(DP, 128)

    def pspec(w):
        return pl.BlockSpec((NC, N, w), lambda i: (0, 0, 0))

    def row(v):
        return v.reshape(1, -1)

    h = x
    for i in range(NUM_LAYERS):
        p = lp[i]
        if i == 0:
            parts = sc_agg_wide(h, src_tw, dst_tw, zeros_half)
            pw = F_IN // NC
        else:
            parts = sc_agg_nar(h, src_tn, dst_tn, zeros_nar)
            pw = DP
        args = (parts, h, p["W1"], row(p["b1"]), row(p["g1"]), row(p["bt1"]),
                p["W2"], row(p["b2"]), row(p["g2"]), row(p["bt2"]))
        nargs = len(args)
        in_specs = [pspec(pw)] + [
            pl.BlockSpec(a.shape, lambda i, nd=a.ndim: (0,) * nd)
            for a in args[1:]]
        if i < NUM_LAYERS - 1:
            h = pl.pallas_call(
                _layer0_body if i == 0 else _layer_body,
                grid=(1,),
                in_specs=in_specs,
                out_specs=pl.BlockSpec((N, DP), lambda i: (0, 0)),
                out_shape=jax.ShapeDtypeStruct((N, DP), jnp.float32),
            )(*args)
        else:
            extra = (node_weight.reshape(N, 1), batch.reshape(N, 1),
                     params["fc_W"], row(params["fc_b"]))
            in_specs += [pl.BlockSpec(a.shape, lambda i, nd=a.ndim: (0,) * nd)
                         for a in extra]
            emb, ge, lg = pl.pallas_call(
                _final_body,
                grid=(1,),
                in_specs=in_specs,
                out_specs=[pl.BlockSpec((N, DP), lambda i: (0, 0)),
                           pl.BlockSpec((G, DP), lambda i: (0, 0)),
                           pl.BlockSpec((G, DP), lambda i: (0, 0))],
                out_shape=[
                    jax.ShapeDtypeStruct((N, DP), jnp.float32),
                    jax.ShapeDtypeStruct((G, DP), jnp.float32),
                    jax.ShapeDtypeStruct((G, DP), jnp.float32),
                ],
            )(*args, *extra)

    node_emb = emb[:, :DIM]
    graph_emb = ge[:, :DIM]
    logits = lg[:, :C]
    return node_emb, graph_emb, logits
